# baseline (speedup 1.0000x reference)
import functools

import jax
import jax.numpy as jnp
from jax import lax
from jax.experimental import pallas as pl
from jax.experimental.pallas import tpu as pltpu
from jax.experimental.pallas import tpu_sc as plsc

_BN = 256


def _argmin_body(xn_ref, a2_ref, cn_ref, b2_ref, idx_ref):
    k = cn_ref.shape[0]
    x = xn_ref[...]
    a2 = a2_ref[...]
    c = cn_ref[...]
    b2 = b2_ref[...]
    t = lax.dot_general(x, c, dimension_numbers=(((1,), (1,)), ((), ())),
                        preferred_element_type=jnp.float32)
    d2 = a2 + b2 - 2.0 * t
    d = jnp.sqrt(jnp.maximum(d2, 0.0))
    mv = jnp.min(d, axis=1, keepdims=True)
    ii = lax.broadcasted_iota(jnp.int32, d.shape, 1)
    idx = jnp.min(jnp.where(d == mv, ii, k), axis=1)
    idx_ref[0, 0, :] = idx


def _tc_argmin(xn, a2, cn, b2):
    n, d = xn.shape
    k = cn.shape[0]
    grid = (n // _BN,)
    out = pl.pallas_call(
        _argmin_body,
        grid=grid,
        in_specs=[
            pl.BlockSpec((_BN, d), lambda i: (i, 0)),
            pl.BlockSpec((_BN, 1), lambda i: (i, 0)),
            pl.BlockSpec((k, d), lambda i: (0, 0)),
            pl.BlockSpec((1, k), lambda i: (0, 0)),
        ],
        out_specs=pl.BlockSpec((1, 1, _BN), lambda i: (i, 0, 0)),
        out_shape=jax.ShapeDtypeStruct((n // _BN, 1, _BN), jnp.int32),
    )(xn, a2, cn, b2)
    return out.reshape(n)


_DP = 128
_CH = 128


def _sc_gather(table_pad, idx):
    n = idx.shape[0]
    info = plsc.get_sparse_core_info()
    nc, ns = info.num_cores, info.num_subcores
    nw = nc * ns
    bpw = n // nw
    nch = bpw // _CH
    mesh = plsc.VectorSubcoreMesh(core_axis_name="c", subcore_axis_name="s")

    @functools.partial(
        pl.kernel, mesh=mesh,
        out_type=jax.ShapeDtypeStruct((n, _DP), jnp.float32),
        scratch_types=[
            pltpu.VMEM((nch, _CH), jnp.int32),
            pltpu.VMEM((nch, _CH, _DP), jnp.float32),
            pltpu.SemaphoreType.DMA,
        ],
    )
    def gk(table_hbm, idx_hbm, out_hbm, idx_v, rows_v, sem):
        wid = lax.axis_index("s") * nc + lax.axis_index("c")
        base = wid * bpw
        pltpu.sync_copy(idx_hbm.at[wid], idx_v)
        descs = [
            pltpu.async_copy(table_hbm.at[idx_v.at[j]], rows_v.at[j], sem)
            for j in range(nch)
        ]
        for j in range(nch):
            descs[j].wait()
            pltpu.sync_copy(rows_v.at[j], out_hbm.at[pl.ds(base + j * _CH, _CH)])

    return gk(table_pad, idx.reshape(nw, nch, _CH))


def kernel(context_repr, cluster_centers):
    xn = context_repr / jnp.maximum(
        jnp.linalg.norm(context_repr, ord=2, axis=1, keepdims=True), 1e-12)
    cn = cluster_centers / jnp.maximum(
        jnp.linalg.norm(cluster_centers, ord=2, axis=1, keepdims=True), 1e-12)
    a2 = jnp.sum(xn * xn, axis=1, keepdims=True)
    b2 = jnp.sum(cn * cn, axis=1)[None, :]
    idx = _tc_argmin(xn, a2, cn, b2)
    k, d = cluster_centers.shape
    table_pad = jnp.pad(cluster_centers, ((0, 0), (0, _DP - d)))
    return _sc_gather(table_pad, idx)[:, :d]
